# Initial kernel scaffold; baseline (speedup 1.0000x reference)
import jax
import jax.numpy as jnp
from jax.experimental import pallas as pl


def kernel(x, table):
    raise NotImplementedError("write your pallas kernel here")


# speedup vs baseline: 9.1252x; 9.1252x over previous
import functools

import jax
import jax.numpy as jnp
from jax import lax
from jax.experimental import pallas as pl
from jax.experimental.pallas import tpu as pltpu
from jax.experimental.pallas import tpu_sc as plsc

VOCAB = 100000
D_MODEL = 128
BATCH = 4096
SEQ = 200

NC, NS = 2, 16
NW = NC * NS
N_ROWS = BATCH * SEQ
PER_W = N_ROWS // NW
CHUNK = 128
N_CHUNKS = PER_W // CHUNK
NBUF = 4
N_GROUPS = N_CHUNKS // NBUF


@functools.partial(
    pl.kernel,
    out_type=jax.ShapeDtypeStruct((N_ROWS, D_MODEL), jnp.float32),
    mesh=plsc.VectorSubcoreMesh(core_axis_name="c", subcore_axis_name="s"),
    scratch_types=[
        pltpu.VMEM((N_CHUNKS, CHUNK), jnp.int32),
        pltpu.VMEM((NBUF, CHUNK, D_MODEL), jnp.float32),
    ] + [pltpu.SemaphoreType.DMA] * NBUF,
)
def _sc_gather(idx_hbm, table_hbm, out_hbm, idx_v, bufs, s0, s1, s2, s3):
    sems = (s0, s1, s2, s3)
    wid = lax.axis_index("s") * NC + lax.axis_index("c")
    row_base = wid * PER_W

    pltpu.sync_copy(idx_hbm.at[wid], idx_v)

    def gather(c, j):
        return pltpu.make_async_copy(
            table_hbm.at[idx_v.at[c]], bufs.at[j], sems[j])

    for j in range(NBUF):
        gather(j, j).start()

    def body(t, carry):
        for j in range(NBUF):
            c = t * NBUF + j
            gather(c, j).wait()
            pltpu.sync_copy(
                bufs.at[j], out_hbm.at[pl.ds(row_base + c * CHUNK, CHUNK)])
            gather(lax.rem(c + NBUF, N_CHUNKS), j).start()
        return carry

    lax.fori_loop(0, N_GROUPS, body, 0)

    for j in range(NBUF):
        gather(j, j).wait()


def kernel(x, table):
    idx = x.astype(jnp.int32).reshape(NW, N_CHUNKS, CHUNK)
    out = _sc_gather(idx, table)
    return out.reshape(BATCH, SEQ, D_MODEL)


# speedup vs baseline: 9.1426x; 1.0019x over previous
import functools

import jax
import jax.numpy as jnp
from jax import lax
from jax.experimental import pallas as pl
from jax.experimental.pallas import tpu as pltpu
from jax.experimental.pallas import tpu_sc as plsc

VOCAB = 100000
D_MODEL = 128
BATCH = 4096
SEQ = 200

NC, NS = 2, 16
NW = NC * NS
N_ROWS = BATCH * SEQ
PER_W = N_ROWS // NW
CHUNK = 64
N_CHUNKS = PER_W // CHUNK
RING = 8
LEAD = RING // 2
N_GROUPS = N_CHUNKS // RING


@functools.partial(
    pl.kernel,
    out_type=jax.ShapeDtypeStruct((N_ROWS, D_MODEL), jnp.float32),
    mesh=plsc.VectorSubcoreMesh(core_axis_name="c", subcore_axis_name="s"),
    scratch_types=[
        pltpu.VMEM((N_CHUNKS, CHUNK), jnp.int32),
        pltpu.VMEM((RING, CHUNK, D_MODEL), jnp.float32),
        pltpu.SemaphoreType.DMA((RING,)),
        pltpu.SemaphoreType.DMA((RING,)),
    ],
)
def _sc_gather(idx_hbm, table_hbm, out_hbm, idx_v, bufs, gsem, ssem):
    wid = lax.axis_index("s") * NC + lax.axis_index("c")
    row_base = wid * PER_W

    pltpu.sync_copy(idx_hbm.at[wid], idx_v)

    def gather(c, j):
        return pltpu.make_async_copy(
            table_hbm.at[idx_v.at[c]], bufs.at[j], gsem.at[j])

    def store(c, j):
        return pltpu.make_async_copy(
            bufs.at[j], out_hbm.at[pl.ds(row_base + c * CHUNK, CHUNK)],
            ssem.at[j])

    def slot(t, j, first_group):
        c = t * RING + j
        jj = (j + LEAD) % RING
        if not (first_group and j < LEAD):
            store(c - LEAD, jj).wait()
        gather(lax.rem(c + LEAD, N_CHUNKS), jj).start()
        gather(c, j).wait()
        store(c, j).start()

    for j in range(LEAD):
        gather(j, j).start()
    for j in range(RING):
        slot(0, j, first_group=True)

    def body(t, carry):
        for j in range(RING):
            slot(t, j, first_group=False)
        return carry

    lax.fori_loop(1, N_GROUPS, body, 0)

    for i in range(LEAD):
        store(N_CHUNKS - LEAD + i, LEAD + i).wait()
        gather(i, i).wait()


def kernel(x, table):
    idx = x.astype(jnp.int32).reshape(NW, N_CHUNKS, CHUNK)
    out = _sc_gather(idx, table)
    return out.reshape(BATCH, SEQ, D_MODEL)
